# baseline (speedup 1.0000x reference)
import jax
import jax.numpy as jnp
from jax import lax
from jax.experimental import pallas as pl
from jax.experimental.pallas import tpu as pltpu
from jax.experimental.pallas import tpu_sc as plsc

_B, _N, _C = 2, 1250, 4
_D = 512
_NODES = _B * _N * _C
_E = 160000
_BM = 1000
_TPAD = 256

_NC = 2
_NS = 16
_NW = _NC * _NS
_EPT = _E // _NW
_NCHUNK = _EPT // 16
_NBKT = 10
_BKT = _NODES // _NBKT
_NBLK = 10
_BLK = 1000
_ACCR = 1024
_TRASH = 1016
_ROWS = 32
_CAP = 5120
_L2CAP = 10080
_BIGSRC = 1 << 20
_HISTR = 80


def _embed_matmul_body(idx_ref, table_ref, w_ref, b_ref, out_ref):
    idx = idx_ref[0, 0, :]
    onehot = (idx[:, None] == lax.broadcasted_iota(jnp.int32, (_BM, _TPAD), 1)).astype(jnp.float32)
    embed = jnp.dot(onehot, table_ref[...], preferred_element_type=jnp.float32)
    out_ref[...] = jnp.dot(embed, w_ref[...], preferred_element_type=jnp.float32) + b_ref[...]


def _embed_matmul(idx_all, table, w, b):
    grid = _NODES // _BM
    idx3 = idx_all.reshape(grid, 1, _BM)
    return pl.pallas_call(
        _embed_matmul_body,
        grid=(grid,),
        in_specs=[
            pl.BlockSpec((1, 1, _BM), lambda i: (i, 0, 0)),
            pl.BlockSpec((_TPAD, _D), lambda i: (0, 0)),
            pl.BlockSpec((_D, _D), lambda i: (0, 0)),
            pl.BlockSpec((1, _D), lambda i: (0, 0)),
        ],
        out_specs=pl.BlockSpec((_BM, _D), lambda i: (i, 0)),
        out_shape=jax.ShapeDtypeStruct((_NODES, _D), jnp.float32),
    )(idx3, table, w, b)


def _relu_matmul_body(x_ref, w_ref, b_ref, out_ref):
    out_ref[...] = jax.nn.relu(
        jnp.dot(x_ref[...], w_ref[...], preferred_element_type=jnp.float32) + b_ref[...]
    )


def _relu_matmul(x, w, b):
    grid = _NODES // _BM
    return pl.pallas_call(
        _relu_matmul_body,
        grid=(grid,),
        in_specs=[
            pl.BlockSpec((_BM, _D), lambda i: (i, 0)),
            pl.BlockSpec((_D, _D), lambda i: (0, 0)),
            pl.BlockSpec((1, _D), lambda i: (0, 0)),
        ],
        out_specs=pl.BlockSpec((_BM, _D), lambda i: (i, 0)),
        out_shape=jax.ShapeDtypeStruct((_NODES, _D), jnp.float32),
    )(x, w, b.reshape(1, _D))


def _matmul_body(x_ref, w_ref, out_ref):
    out_ref[...] = jnp.dot(x_ref[...], w_ref[...], preferred_element_type=jnp.float32)


def _matmul(x, w):
    grid = _NODES // _BM
    return pl.pallas_call(
        _matmul_body,
        grid=(grid,),
        in_specs=[
            pl.BlockSpec((_BM, _D), lambda i: (i, 0)),
            pl.BlockSpec((_D, _D), lambda i: (0, 0)),
        ],
        out_specs=pl.BlockSpec((_BM, _D), lambda i: (i, 0)),
        out_shape=jax.ShapeDtypeStruct((_NODES, _D), jnp.float32),
    )(x, w)


def _out_norm_body(p_ref, cinv_ref, yr_ref, wl_ref, bl_ref, out_ref):
    mean = p_ref[...] * cinv_ref[...]
    out = (
        jnp.dot(mean, wl_ref[...], preferred_element_type=jnp.float32)
        + bl_ref[...]
        + yr_ref[...]
    )
    ssq = jnp.sum(out * out, axis=-1, keepdims=True)
    out_ref[...] = out / jnp.maximum(jnp.sqrt(ssq), 1e-12)


def _out_norm(part, cnt_inv, yr, wl, bl):
    grid = _NODES // _BM
    return pl.pallas_call(
        _out_norm_body,
        grid=(grid,),
        in_specs=[
            pl.BlockSpec((_BM, _D), lambda i: (i, 0)),
            pl.BlockSpec((_BM, 1), lambda i: (i, 0)),
            pl.BlockSpec((_BM, _D), lambda i: (i, 0)),
            pl.BlockSpec((_D, _D), lambda i: (0, 0)),
            pl.BlockSpec((1, _D), lambda i: (0, 0)),
        ],
        out_specs=pl.BlockSpec((_BM, _D), lambda i: (i, 0)),
        out_shape=jax.ShapeDtypeStruct((_NODES, _D), jnp.float32),
    )(part, cnt_inv, yr, wl, bl.reshape(1, _D))


_SC_MESH = plsc.VectorSubcoreMesh(core_axis_name="c", subcore_axis_name="s")
_SC_PARAMS = pltpu.CompilerParams(use_tc_tiling_on_sc=False,
                                  needs_layout_passes=False)


def _prep_body(src_hbm, dst_hbm, cnt_hbm, glist_hbm, nch_hbm,
               src_v, dst_v, hist_v, lsrc_v, ldst_v, cnts_v, cnt_sh, sem):
    c = lax.axis_index("c")
    s = lax.axis_index("s")
    wid = s * _NC + c
    pltpu.sync_copy(src_hbm.at[pl.ds(wid * _EPT, _EPT)], src_v.at[pl.ds(0, _EPT)])
    pltpu.sync_copy(dst_hbm.at[pl.ds(wid * _EPT, _EPT)], dst_v.at[pl.ds(0, _EPT)])

    zero16 = jnp.zeros((16,), jnp.float32)
    lanes = lax.iota(jnp.int32, 16)

    def zero_hist(t, carry):
        hist_v[t // 8, pl.ds((t % 8) * 16, 16)] = zero16
        return carry
    lax.fori_loop(0, _HISTR * 8, zero_hist, 0)

    @pl.when(s < 10)
    def _zero_sh():
        pltpu.sync_copy(hist_v.at[pl.ds(s * 8, 8)], cnt_sh.at[pl.ds(s * 8, 8)])
    plsc.subcore_barrier()

    ones = jnp.ones((16,), jnp.float32)

    def count(i, carry):
        nlanes = jnp.where(i == _NCHUNK, 8, 16)
        m = lanes < nlanes
        d = dst_v[pl.ds(i * 16, 16)]
        plsc.addupdate_scatter(hist_v, [lax.shift_right_logical(d, 7), d & 127],
                               ones, mask=m)
        return carry
    lax.fori_loop(0, _NCHUNK + 1, count, 0)

    for k in range(_HISTR // 16):
        idxv = lax.iota(jnp.int32, 16) + k * 16
        pltpu.sync_copy(hist_v.at[pl.ds(k * 16, 16)], cnt_sh.at[idxv], add=True)
    plsc.subcore_barrier()

    @pl.when(s < 10)
    def _writeback():
        pltpu.sync_copy(cnt_sh.at[pl.ds(s * 8, 8)], cnt_hbm.at[c, pl.ds(s * 8, 8)])

    def filt(i, ns):
        nlanes = jnp.where(i == _NCHUNK, 8, 16)
        m = lanes < nlanes
        d = dst_v[pl.ds(i * 16, 16)]
        sv = src_v[pl.ds(i * 16, 16)]
        out = []
        for q in range(_NBKT):
            inb = m & (d >= q * _BKT) & (d < (q + 1) * _BKT)
            plsc.store_compressed(lsrc_v.at[pl.ds(q * _CAP + ns[q], 16)], sv, mask=inb)
            plsc.store_compressed(ldst_v.at[pl.ds(q * _CAP + ns[q], 16)], d - q * _BKT,
                                  mask=inb)
            out.append(ns[q] + jnp.sum(inb.astype(jnp.int32)))
        return tuple(out)
    ns = lax.fori_loop(0, _NCHUNK + 1, filt, (0,) * _NBKT)

    trash16 = jnp.full((16,), _TRASH, jnp.int32)
    big16 = jnp.full((16,), _BIGSRC, jnp.int32)
    for q in range(_NBKT):
        for t in range(2):
            lsrc_v[pl.ds(q * _CAP + ns[q] + t * 16, 16)] = big16
            ldst_v[pl.ds(q * _CAP + ns[q] + t * 16, 16)] = trash16
        n1ch = (ns[q] + 15) // 16
        cnts_v[pl.ds(q * 16, 16)] = jnp.full((16,), n1ch, jnp.int32)
        pltpu.sync_copy(lsrc_v.at[pl.ds(q * _CAP, _CAP)], glist_hbm.at[wid, q, 0])
        pltpu.sync_copy(ldst_v.at[pl.ds(q * _CAP, _CAP)], glist_hbm.at[wid, q, 1])
    pltpu.sync_copy(cnts_v, nch_hbm.at[wid])


def _sc_prep(src, dst):
    return pl.kernel(
        _prep_body,
        out_type=(
            jax.ShapeDtypeStruct((_NC, _HISTR, 128), jnp.float32),
            jax.ShapeDtypeStruct((_NW, _NBKT, 2, _CAP), jnp.int32),
            jax.ShapeDtypeStruct((_NW, _NBKT * 16), jnp.int32),
        ),
        mesh=_SC_MESH,
        compiler_params=_SC_PARAMS,
        scratch_types=[
            pltpu.VMEM((_EPT + 16,), jnp.int32),
            pltpu.VMEM((_EPT + 16,), jnp.int32),
            pltpu.VMEM((_HISTR, 128), jnp.float32),
            pltpu.VMEM((_NBKT * _CAP,), jnp.int32),
            pltpu.VMEM((_NBKT * _CAP,), jnp.int32),
            pltpu.VMEM((_NBKT * 16,), jnp.int32),
            pltpu.VMEM_SHARED((_HISTR, 128), jnp.float32),
            pltpu.SemaphoreType.DMA,
        ],
    )(src, dst)


def _segsum_body(xp_hbm, glist_hbm, nch_hbm, zrows_hbm, out_hbm,
                 l1_v, l2s_v, l2d_v, rows_a, rows_b, didx_a, didx_b,
                 cbuf_v, stage_sh, acc_sh, sem_a, sem_b):
    c = lax.axis_index("c")
    s = lax.axis_index("s")
    pltpu.sync_copy(nch_hbm.at[2 * s], cbuf_v.at[pl.ds(0, _NBKT * 16)])
    pltpu.sync_copy(nch_hbm.at[2 * s + 1], cbuf_v.at[pl.ds(_NBKT * 16, _NBKT * 16)])

    lanes = lax.iota(jnp.int32, 16)
    lane0 = (lanes == 0).astype(jnp.int32)
    trash16 = jnp.full((16,), _TRASH, jnp.int32)
    zero16i = jnp.zeros((16,), jnp.int32)

    def bucket(qq, carry_q):
        q = c * (_NBKT // _NC) + qq
        lo = q * _BKT

        for t in range(8):
            pltpu.sync_copy(zrows_hbm, acc_sh.at[pl.ds(s * 64 + t * 8, 8)])
        plsc.subcore_barrier()

        def block(b, carry_b):
            @pl.when(s < 15)
            def _stage():
                pltpu.sync_copy(xp_hbm.at[pl.ds(b * _BLK + s * 64, 64)],
                                stage_sh.at[pl.ds(s * 64, 64)])

            @pl.when(s == 15)
            def _stage_tail():
                pltpu.sync_copy(xp_hbm.at[pl.ds(b * _BLK + 960, 40)],
                                stage_sh.at[pl.ds(960, 40)])
            plsc.subcore_barrier()

            n2 = 0
            for pi in range(2):
                pt = 2 * s + pi
                pltpu.sync_copy(glist_hbm.at[pt, q], l1_v)
                n1ch = jnp.sum(cbuf_v[pl.ds(pi * _NBKT * 16 + q * 16, 16)] * lane0)

                def filt(i, n):
                    sv = l1_v[0, pl.ds(i * 16, 16)]
                    dv = l1_v[1, pl.ds(i * 16, 16)]
                    inb = (sv >= b * _BLK) & (sv < (b + 1) * _BLK)
                    plsc.store_compressed(l2s_v.at[pl.ds(n, 16)], sv - b * _BLK,
                                          mask=inb)
                    plsc.store_compressed(l2d_v.at[pl.ds(n, 16)], dv, mask=inb)
                    return n + jnp.sum(inb.astype(jnp.int32))
                n2 = lax.fori_loop(0, n1ch, filt, n2)

            for t in range(3):
                l2s_v[pl.ds(n2 + t * 16, 16)] = zero16i
                l2d_v[pl.ds(n2 + t * 16, 16)] = trash16
            nch2 = (n2 + _ROWS - 1) // _ROWS

            @pl.when(nch2 > 0)
            def _prime0():
                pltpu.async_copy(stage_sh.at[l2s_v.at[pl.ds(0, _ROWS)]],
                                 rows_a, sem_a)

            @pl.when(nch2 > 1)
            def _prime1():
                pltpu.async_copy(stage_sh.at[l2s_v.at[pl.ds(_ROWS, _ROWS)]],
                                 rows_b, sem_b)

            def chunk(j, carry):
                nxt = j + 2

                @pl.when(j % 2 == 0)
                def _even():
                    pltpu.make_async_copy(
                        stage_sh.at[l2s_v.at[pl.ds(j * _ROWS, _ROWS)]],
                        rows_a, sem_a).wait()
                    for k in range(_ROWS // 16):
                        didx_a[pl.ds(k * 16, 16)] = l2d_v[pl.ds(j * _ROWS + k * 16, 16)]
                    pltpu.sync_copy(rows_a, acc_sh.at[didx_a], add=True)

                    @pl.when(nxt < nch2)
                    def _issue():
                        pltpu.async_copy(
                            stage_sh.at[l2s_v.at[pl.ds(nxt * _ROWS, _ROWS)]],
                            rows_a, sem_a)

                @pl.when(j % 2 == 1)
                def _odd():
                    pltpu.make_async_copy(
                        stage_sh.at[l2s_v.at[pl.ds(j * _ROWS, _ROWS)]],
                        rows_b, sem_b).wait()
                    for k in range(_ROWS // 16):
                        didx_b[pl.ds(k * 16, 16)] = l2d_v[pl.ds(j * _ROWS + k * 16, 16)]
                    pltpu.sync_copy(rows_b, acc_sh.at[didx_b], add=True)

                    @pl.when(nxt < nch2)
                    def _issue():
                        pltpu.async_copy(
                            stage_sh.at[l2s_v.at[pl.ds(nxt * _ROWS, _ROWS)]],
                            rows_b, sem_b)
                return carry
            lax.fori_loop(0, nch2, chunk, 0)
            plsc.subcore_barrier()
            return carry_b
        lax.fori_loop(0, _NBLK, block, 0)

        @pl.when(s < 15)
        def _wb():
            pltpu.sync_copy(acc_sh.at[pl.ds(s * 64, 64)],
                            out_hbm.at[pl.ds(lo + s * 64, 64)])

        @pl.when(s == 15)
        def _wb_tail():
            pltpu.sync_copy(acc_sh.at[pl.ds(960, 40)],
                            out_hbm.at[pl.ds(lo + 960, 40)])
        plsc.subcore_barrier()
        return carry_q
    lax.fori_loop(0, _NBKT // _NC, bucket, 0)


def _sc_segsum(xp, glist, nch, zrows):
    return pl.kernel(
        _segsum_body,
        out_type=jax.ShapeDtypeStruct((_NODES, _D), jnp.float32),
        mesh=_SC_MESH,
        compiler_params=_SC_PARAMS,
        scratch_types=[
            pltpu.VMEM((2, _CAP), jnp.int32),
            pltpu.VMEM((_L2CAP,), jnp.int32),
            pltpu.VMEM((_L2CAP,), jnp.int32),
            pltpu.VMEM((_ROWS, _D), jnp.float32),
            pltpu.VMEM((_ROWS, _D), jnp.float32),
            pltpu.VMEM((_ROWS,), jnp.int32),
            pltpu.VMEM((_ROWS,), jnp.int32),
            pltpu.VMEM((2 * _NBKT * 16,), jnp.int32),
            pltpu.VMEM_SHARED((_BLK, _D), jnp.float32),
            pltpu.VMEM_SHARED((_ACCR, _D), jnp.float32),
            pltpu.SemaphoreType.DMA,
            pltpu.SemaphoreType.DMA,
        ],
    )(xp, glist, nch, zrows)


def kernel(svg_path, svg_path_mask, edge_index, type_embed, coor_embed, W_in, b_in,
           W1p, b1p, W1l, b1l, W1r, W2p, b2p, W2l, b2l, W2r):
    svg = jnp.where(svg_path_mask, svg_path, 0)
    cmd_idx = svg[:, :, 0]
    coor_idx = svg[:, :, 1:] + 3
    idx_all = jnp.concatenate(
        [cmd_idx.reshape(_B, _N), coor_idx.reshape(_B, _N * (_C - 1))], axis=1
    ).reshape(_NODES)
    table = jnp.concatenate(
        [type_embed, coor_embed,
         jnp.zeros((_TPAD - 3 - 200, _D), jnp.float32)], axis=0
    )
    src = edge_index[0]
    dst = edge_index[1]
    zrows = jnp.zeros((8, _D), jnp.float32)

    cntp, glistw, nchw = _sc_prep(src, dst)
    cnt = (cntp[0] + cntp[1]).reshape(_HISTR * 128)[:_NODES]
    cnt_inv = (1.0 / jnp.maximum(cnt, 1.0)).reshape(_NODES, 1)

    x = _embed_matmul(idx_all, table, W_in, b_in.reshape(1, _D))

    xp1 = _relu_matmul(x, W1p, b1p)
    yr1 = _matmul(xp1, W1r)
    part1 = _sc_segsum(xp1, glistw, nchw, zrows)
    x1 = _out_norm(part1, cnt_inv, yr1, W1l, b1l)

    xp2 = _relu_matmul(x1, W2p, b2p)
    yr2 = _matmul(xp2, W2r)
    part2 = _sc_segsum(xp2, glistw, nchw, zrows)
    x2 = _out_norm(part2, cnt_inv, yr2, W2l, b2l)
    return x2


# speedup vs baseline: 1.6226x; 1.6226x over previous
import jax
import jax.numpy as jnp
from jax import lax
from jax.experimental import pallas as pl
from jax.experimental.pallas import tpu as pltpu
from jax.experimental.pallas import tpu_sc as plsc

_B, _N, _C = 2, 1250, 4
_D = 512
_NODES = _B * _N * _C
_E = 160000
_BM = 2000
_TPAD = 256

_NC = 2
_NS = 16
_NW = _NC * _NS
_EPT = _E // _NW
_NCHUNK = _EPT // 16
_NBKT = 10
_BKT = _NODES // _NBKT
_NBLK = 5
_BLK = 2000
_ACCR = 1024
_TRASH = 1016
_ROWS = 64
_CAP = 5120
_L2CAP = 10176
_BIGSRC = 1 << 20
_HISTR = 80


def _embed_matmul_body(idx_ref, table_ref, w_ref, b_ref, out_ref):
    idx = idx_ref[0, 0, :]
    onehot = (idx[:, None] == lax.broadcasted_iota(jnp.int32, (_BM, _TPAD), 1)).astype(jnp.float32)
    embed = jnp.dot(onehot, table_ref[...], preferred_element_type=jnp.float32)
    out_ref[...] = jnp.dot(embed, w_ref[...], preferred_element_type=jnp.float32) + b_ref[...]


def _embed_matmul(idx_all, table, w, b):
    grid = _NODES // _BM
    idx3 = idx_all.reshape(grid, 1, _BM)
    return pl.pallas_call(
        _embed_matmul_body,
        grid=(grid,),
        in_specs=[
            pl.BlockSpec((1, 1, _BM), lambda i: (i, 0, 0)),
            pl.BlockSpec((_TPAD, _D), lambda i: (0, 0)),
            pl.BlockSpec((_D, _D), lambda i: (0, 0)),
            pl.BlockSpec((1, _D), lambda i: (0, 0)),
        ],
        out_specs=pl.BlockSpec((_BM, _D), lambda i: (i, 0)),
        out_shape=jax.ShapeDtypeStruct((_NODES, _D), jnp.float32),
    )(idx3, table, w, b)


def _relu_matmul_body(x_ref, w_ref, b_ref, out_ref, outb_ref):
    xp = jax.nn.relu(
        jnp.dot(x_ref[...], w_ref[...], preferred_element_type=jnp.float32) + b_ref[...]
    )
    out_ref[...] = xp
    outb_ref[...] = xp.astype(jnp.bfloat16)


def _relu_matmul(x, w, b):
    grid = _NODES // _BM
    return pl.pallas_call(
        _relu_matmul_body,
        grid=(grid,),
        in_specs=[
            pl.BlockSpec((_BM, _D), lambda i: (i, 0)),
            pl.BlockSpec((_D, _D), lambda i: (0, 0)),
            pl.BlockSpec((1, _D), lambda i: (0, 0)),
        ],
        out_specs=[
            pl.BlockSpec((_BM, _D), lambda i: (i, 0)),
            pl.BlockSpec((_BM, _D), lambda i: (i, 0)),
        ],
        out_shape=[
            jax.ShapeDtypeStruct((_NODES, _D), jnp.float32),
            jax.ShapeDtypeStruct((_NODES, _D), jnp.bfloat16),
        ],
    )(x, w, b.reshape(1, _D))


def _matmul_body(x_ref, w_ref, out_ref):
    out_ref[...] = jnp.dot(x_ref[...], w_ref[...], preferred_element_type=jnp.float32)


def _matmul(x, w):
    grid = _NODES // _BM
    return pl.pallas_call(
        _matmul_body,
        grid=(grid,),
        in_specs=[
            pl.BlockSpec((_BM, _D), lambda i: (i, 0)),
            pl.BlockSpec((_D, _D), lambda i: (0, 0)),
        ],
        out_specs=pl.BlockSpec((_BM, _D), lambda i: (i, 0)),
        out_shape=jax.ShapeDtypeStruct((_NODES, _D), jnp.float32),
    )(x, w)


def _out_norm_body(p_ref, cinv_ref, yr_ref, wl_ref, bl_ref, out_ref):
    mean = p_ref[...].astype(jnp.float32) * cinv_ref[...]
    out = (
        jnp.dot(mean, wl_ref[...], preferred_element_type=jnp.float32)
        + bl_ref[...]
        + yr_ref[...]
    )
    ssq = jnp.sum(out * out, axis=-1, keepdims=True)
    out_ref[...] = out / jnp.maximum(jnp.sqrt(ssq), 1e-12)


def _out_norm(part, cnt_inv, yr, wl, bl):
    grid = _NODES // _BM
    return pl.pallas_call(
        _out_norm_body,
        grid=(grid,),
        in_specs=[
            pl.BlockSpec((_BM, _D), lambda i: (i, 0)),
            pl.BlockSpec((_BM, 1), lambda i: (i, 0)),
            pl.BlockSpec((_BM, _D), lambda i: (i, 0)),
            pl.BlockSpec((_D, _D), lambda i: (0, 0)),
            pl.BlockSpec((1, _D), lambda i: (0, 0)),
        ],
        out_specs=pl.BlockSpec((_BM, _D), lambda i: (i, 0)),
        out_shape=jax.ShapeDtypeStruct((_NODES, _D), jnp.float32),
    )(part, cnt_inv, yr, wl, bl.reshape(1, _D))


_SC_MESH = plsc.VectorSubcoreMesh(core_axis_name="c", subcore_axis_name="s")
_SC_PARAMS = pltpu.CompilerParams(use_tc_tiling_on_sc=False,
                                  needs_layout_passes=False)


def _prep_body(src_hbm, dst_hbm, cnt_hbm, glist_hbm, nch_hbm,
               src_v, dst_v, hist_v, lsrc_v, ldst_v, cnts_v, cnt_sh, sem):
    c = lax.axis_index("c")
    s = lax.axis_index("s")
    wid = s * _NC + c
    pltpu.sync_copy(src_hbm.at[pl.ds(wid * _EPT, _EPT)], src_v.at[pl.ds(0, _EPT)])
    pltpu.sync_copy(dst_hbm.at[pl.ds(wid * _EPT, _EPT)], dst_v.at[pl.ds(0, _EPT)])

    zero16 = jnp.zeros((16,), jnp.float32)
    lanes = lax.iota(jnp.int32, 16)

    def zero_hist(t, carry):
        hist_v[t // 8, pl.ds((t % 8) * 16, 16)] = zero16
        return carry
    lax.fori_loop(0, _HISTR * 8, zero_hist, 0)

    @pl.when(s < 10)
    def _zero_sh():
        pltpu.sync_copy(hist_v.at[pl.ds(s * 8, 8)], cnt_sh.at[pl.ds(s * 8, 8)])
    plsc.subcore_barrier()

    ones = jnp.ones((16,), jnp.float32)

    def count(i, carry):
        nlanes = jnp.where(i == _NCHUNK, 8, 16)
        m = lanes < nlanes
        d = dst_v[pl.ds(i * 16, 16)]
        plsc.addupdate_scatter(hist_v, [lax.shift_right_logical(d, 7), d & 127],
                               ones, mask=m)
        return carry
    lax.fori_loop(0, _NCHUNK + 1, count, 0)

    for k in range(_HISTR // 16):
        idxv = lax.iota(jnp.int32, 16) + k * 16
        pltpu.sync_copy(hist_v.at[pl.ds(k * 16, 16)], cnt_sh.at[idxv], add=True)
    plsc.subcore_barrier()

    @pl.when(s < 10)
    def _writeback():
        pltpu.sync_copy(cnt_sh.at[pl.ds(s * 8, 8)], cnt_hbm.at[c, pl.ds(s * 8, 8)])

    def filt(i, ns):
        nlanes = jnp.where(i == _NCHUNK, 8, 16)
        m = lanes < nlanes
        d = dst_v[pl.ds(i * 16, 16)]
        sv = src_v[pl.ds(i * 16, 16)]
        out = []
        for q in range(_NBKT):
            inb = m & (d >= q * _BKT) & (d < (q + 1) * _BKT)
            plsc.store_compressed(lsrc_v.at[pl.ds(q * _CAP + ns[q], 16)], sv, mask=inb)
            plsc.store_compressed(ldst_v.at[pl.ds(q * _CAP + ns[q], 16)], d - q * _BKT,
                                  mask=inb)
            out.append(ns[q] + jnp.sum(inb.astype(jnp.int32)))
        return tuple(out)
    ns = lax.fori_loop(0, _NCHUNK + 1, filt, (0,) * _NBKT)

    trash16 = jnp.full((16,), _TRASH, jnp.int32)
    big16 = jnp.full((16,), _BIGSRC, jnp.int32)
    for q in range(_NBKT):
        for t in range(2):
            lsrc_v[pl.ds(q * _CAP + ns[q] + t * 16, 16)] = big16
            ldst_v[pl.ds(q * _CAP + ns[q] + t * 16, 16)] = trash16
        n1ch = (ns[q] + 15) // 16
        cnts_v[pl.ds(q * 16, 16)] = jnp.full((16,), n1ch, jnp.int32)
        pltpu.sync_copy(lsrc_v.at[pl.ds(q * _CAP, _CAP)], glist_hbm.at[wid, q, 0])
        pltpu.sync_copy(ldst_v.at[pl.ds(q * _CAP, _CAP)], glist_hbm.at[wid, q, 1])
    pltpu.sync_copy(cnts_v, nch_hbm.at[wid])


def _sc_prep(src, dst):
    return pl.kernel(
        _prep_body,
        out_type=(
            jax.ShapeDtypeStruct((_NC, _HISTR, 128), jnp.float32),
            jax.ShapeDtypeStruct((_NW, _NBKT, 2, _CAP), jnp.int32),
            jax.ShapeDtypeStruct((_NW, _NBKT * 16), jnp.int32),
        ),
        mesh=_SC_MESH,
        compiler_params=_SC_PARAMS,
        scratch_types=[
            pltpu.VMEM((_EPT + 16,), jnp.int32),
            pltpu.VMEM((_EPT + 16,), jnp.int32),
            pltpu.VMEM((_HISTR, 128), jnp.float32),
            pltpu.VMEM((_NBKT * _CAP,), jnp.int32),
            pltpu.VMEM((_NBKT * _CAP,), jnp.int32),
            pltpu.VMEM((_NBKT * 16,), jnp.int32),
            pltpu.VMEM_SHARED((_HISTR, 128), jnp.float32),
            pltpu.SemaphoreType.DMA,
        ],
    )(src, dst)


def _segsum_body(xp_hbm, glist_hbm, nch_hbm, zrows_hbm, out_hbm,
                 l1_v, l2s_v, l2d_v, rows_a, rows_b, didx_a, didx_b,
                 cbuf_v, stage_sh, acc_sh, sem_a, sem_b):
    c = lax.axis_index("c")
    s = lax.axis_index("s")
    pltpu.sync_copy(nch_hbm.at[2 * s], cbuf_v.at[pl.ds(0, _NBKT * 16)])
    pltpu.sync_copy(nch_hbm.at[2 * s + 1], cbuf_v.at[pl.ds(_NBKT * 16, _NBKT * 16)])

    lanes = lax.iota(jnp.int32, 16)
    lane0 = (lanes == 0).astype(jnp.int32)
    trash16 = jnp.full((16,), _TRASH, jnp.int32)
    zero16i = jnp.zeros((16,), jnp.int32)

    def bucket(qq, carry_q):
        q = c * (_NBKT // _NC) + qq
        lo = q * _BKT

        pltpu.sync_copy(glist_hbm.at[2 * s, q], l1_v.at[0])
        pltpu.sync_copy(glist_hbm.at[2 * s + 1, q], l1_v.at[1])

        for t in range(8):
            pltpu.sync_copy(zrows_hbm, acc_sh.at[pl.ds(s * 64 + t * 8, 8)])
        plsc.subcore_barrier()

        def block(b, carry_b):
            @pl.when(s < 15)
            def _stage():
                pltpu.sync_copy(xp_hbm.at[pl.ds(b * _BLK + s * 128, 128)],
                                stage_sh.at[pl.ds(s * 128, 128)])

            @pl.when(s == 15)
            def _stage_tail():
                pltpu.sync_copy(xp_hbm.at[pl.ds(b * _BLK + 1920, 80)],
                                stage_sh.at[pl.ds(1920, 80)])
            plsc.subcore_barrier()

            n2 = 0
            for pi in range(2):
                n1ch = jnp.sum(cbuf_v[pl.ds(pi * _NBKT * 16 + q * 16, 16)] * lane0)

                def filt(i, n):
                    sv = l1_v[pi, 0, pl.ds(i * 16, 16)]
                    dv = l1_v[pi, 1, pl.ds(i * 16, 16)]
                    inb = (sv >= b * _BLK) & (sv < (b + 1) * _BLK)
                    plsc.store_compressed(l2s_v.at[pl.ds(n, 16)], sv - b * _BLK,
                                          mask=inb)
                    plsc.store_compressed(l2d_v.at[pl.ds(n, 16)], dv, mask=inb)
                    return n + jnp.sum(inb.astype(jnp.int32))
                n2 = lax.fori_loop(0, n1ch, filt, n2)

            for t in range(5):
                l2s_v[pl.ds(n2 + t * 16, 16)] = zero16i
                l2d_v[pl.ds(n2 + t * 16, 16)] = trash16
            nch2 = (n2 + _ROWS - 1) // _ROWS

            @pl.when(nch2 > 0)
            def _prime0():
                pltpu.async_copy(stage_sh.at[l2s_v.at[pl.ds(0, _ROWS)]],
                                 rows_a, sem_a)

            @pl.when(nch2 > 1)
            def _prime1():
                pltpu.async_copy(stage_sh.at[l2s_v.at[pl.ds(_ROWS, _ROWS)]],
                                 rows_b, sem_b)

            def chunk(j, carry):
                nxt = j + 2

                @pl.when(j % 2 == 0)
                def _even():
                    pltpu.make_async_copy(
                        stage_sh.at[l2s_v.at[pl.ds(j * _ROWS, _ROWS)]],
                        rows_a, sem_a).wait()
                    for k in range(_ROWS // 16):
                        didx_a[pl.ds(k * 16, 16)] = l2d_v[pl.ds(j * _ROWS + k * 16, 16)]
                    pltpu.sync_copy(rows_a, acc_sh.at[didx_a], add=True)

                    @pl.when(nxt < nch2)
                    def _issue():
                        pltpu.async_copy(
                            stage_sh.at[l2s_v.at[pl.ds(nxt * _ROWS, _ROWS)]],
                            rows_a, sem_a)

                @pl.when(j % 2 == 1)
                def _odd():
                    pltpu.make_async_copy(
                        stage_sh.at[l2s_v.at[pl.ds(j * _ROWS, _ROWS)]],
                        rows_b, sem_b).wait()
                    for k in range(_ROWS // 16):
                        didx_b[pl.ds(k * 16, 16)] = l2d_v[pl.ds(j * _ROWS + k * 16, 16)]
                    pltpu.sync_copy(rows_b, acc_sh.at[didx_b], add=True)

                    @pl.when(nxt < nch2)
                    def _issue():
                        pltpu.async_copy(
                            stage_sh.at[l2s_v.at[pl.ds(nxt * _ROWS, _ROWS)]],
                            rows_b, sem_b)
                return carry
            lax.fori_loop(0, nch2, chunk, 0)
            plsc.subcore_barrier()
            return carry_b
        lax.fori_loop(0, _NBLK, block, 0)

        @pl.when(s < 15)
        def _wb():
            pltpu.sync_copy(acc_sh.at[pl.ds(s * 64, 64)],
                            out_hbm.at[pl.ds(lo + s * 64, 64)])

        @pl.when(s == 15)
        def _wb_tail():
            pltpu.sync_copy(acc_sh.at[pl.ds(960, 40)],
                            out_hbm.at[pl.ds(lo + 960, 40)])
        plsc.subcore_barrier()
        return carry_q
    lax.fori_loop(0, _NBKT // _NC, bucket, 0)


def _sc_segsum(xp, glist, nch, zrows):
    return pl.kernel(
        _segsum_body,
        out_type=jax.ShapeDtypeStruct((_NODES, _D), jnp.bfloat16),
        mesh=_SC_MESH,
        compiler_params=_SC_PARAMS,
        scratch_types=[
            pltpu.VMEM((2, 2, _CAP), jnp.int32),
            pltpu.VMEM((_L2CAP,), jnp.int32),
            pltpu.VMEM((_L2CAP,), jnp.int32),
            pltpu.VMEM((_ROWS, _D), jnp.bfloat16),
            pltpu.VMEM((_ROWS, _D), jnp.bfloat16),
            pltpu.VMEM((_ROWS,), jnp.int32),
            pltpu.VMEM((_ROWS,), jnp.int32),
            pltpu.VMEM((2 * _NBKT * 16,), jnp.int32),
            pltpu.VMEM_SHARED((_BLK, _D), jnp.bfloat16),
            pltpu.VMEM_SHARED((_ACCR, _D), jnp.bfloat16),
            pltpu.SemaphoreType.DMA,
            pltpu.SemaphoreType.DMA,
        ],
    )(xp, glist, nch, zrows)


def kernel(svg_path, svg_path_mask, edge_index, type_embed, coor_embed, W_in, b_in,
           W1p, b1p, W1l, b1l, W1r, W2p, b2p, W2l, b2l, W2r):
    svg = jnp.where(svg_path_mask, svg_path, 0)
    cmd_idx = svg[:, :, 0]
    coor_idx = svg[:, :, 1:] + 3
    idx_all = jnp.concatenate(
        [cmd_idx.reshape(_B, _N), coor_idx.reshape(_B, _N * (_C - 1))], axis=1
    ).reshape(_NODES)
    table = jnp.concatenate(
        [type_embed, coor_embed,
         jnp.zeros((_TPAD - 3 - 200, _D), jnp.float32)], axis=0
    )
    src = edge_index[0]
    dst = edge_index[1]
    zrows = jnp.zeros((8, _D), jnp.bfloat16)

    cntp, glistw, nchw = _sc_prep(src, dst)
    cnt = (cntp[0] + cntp[1]).reshape(_HISTR * 128)[:_NODES]
    cnt_inv = (1.0 / jnp.maximum(cnt, 1.0)).reshape(_NODES, 1)

    x = _embed_matmul(idx_all, table, W_in, b_in.reshape(1, _D))

    xp1, xpb1 = _relu_matmul(x, W1p, b1p)
    yr1 = _matmul(xp1, W1r)
    part1 = _sc_segsum(xpb1, glistw, nchw, zrows)
    x1 = _out_norm(part1, cnt_inv, yr1, W1l, b1l)

    xp2, xpb2 = _relu_matmul(x1, W2p, b2p)
    yr2 = _matmul(xp2, W2r)
    part2 = _sc_segsum(xpb2, glistw, nchw, zrows)
    x2 = _out_norm(part2, cnt_inv, yr2, W2l, b2l)
    return x2


# speedup vs baseline: 1.6504x; 1.0171x over previous
import jax
import jax.numpy as jnp
from jax import lax
from jax.experimental import pallas as pl
from jax.experimental.pallas import tpu as pltpu
from jax.experimental.pallas import tpu_sc as plsc

_B, _N, _C = 2, 1250, 4
_D = 512
_NODES = _B * _N * _C
_E = 160000
_BM = 2000
_TPAD = 256

_NC = 2
_NS = 16
_NW = _NC * _NS
_EPT = _E // _NW
_NCHUNK = _EPT // 16
_NBKT = 10
_BKT = _NODES // _NBKT
_NBLK = 5
_BLK = 2000
_ACCR = 1024
_TRASH = 1016
_ROWS = 64
_CAP = 5120
_L2CAP = 10176
_BIGSRC = 1 << 20
_HISTR = 80


def _embed_matmul_body(idx_ref, table_ref, w_ref, b_ref, out_ref):
    idx = idx_ref[0, 0, :]
    onehot = (idx[:, None] == lax.broadcasted_iota(jnp.int32, (_BM, _TPAD), 1)).astype(jnp.bfloat16)
    embed = jnp.dot(onehot, table_ref[...], preferred_element_type=jnp.float32)
    h = jnp.dot(embed.astype(jnp.bfloat16), w_ref[...],
                preferred_element_type=jnp.float32) + b_ref[...]
    out_ref[...] = h.astype(jnp.bfloat16)


def _embed_matmul(idx_all, table, w, b):
    grid = _NODES // _BM
    idx3 = idx_all.reshape(grid, 1, _BM)
    return pl.pallas_call(
        _embed_matmul_body,
        grid=(grid,),
        in_specs=[
            pl.BlockSpec((1, 1, _BM), lambda i: (i, 0, 0)),
            pl.BlockSpec((_TPAD, _D), lambda i: (0, 0)),
            pl.BlockSpec((_D, _D), lambda i: (0, 0)),
            pl.BlockSpec((1, _D), lambda i: (0, 0)),
        ],
        out_specs=pl.BlockSpec((_BM, _D), lambda i: (i, 0)),
        out_shape=jax.ShapeDtypeStruct((_NODES, _D), jnp.bfloat16),
    )(idx3, table, w, b)


def _relu_matmul_body(x_ref, w_ref, b_ref, out_ref):
    xp = jax.nn.relu(
        jnp.dot(x_ref[...], w_ref[...], preferred_element_type=jnp.float32) + b_ref[...]
    )
    out_ref[...] = xp.astype(jnp.bfloat16)


def _relu_matmul(x, w, b):
    grid = _NODES // _BM
    return pl.pallas_call(
        _relu_matmul_body,
        grid=(grid,),
        in_specs=[
            pl.BlockSpec((_BM, _D), lambda i: (i, 0)),
            pl.BlockSpec((_D, _D), lambda i: (0, 0)),
            pl.BlockSpec((1, _D), lambda i: (0, 0)),
        ],
        out_specs=pl.BlockSpec((_BM, _D), lambda i: (i, 0)),
        out_shape=jax.ShapeDtypeStruct((_NODES, _D), jnp.bfloat16),
    )(x, w, b.reshape(1, _D))


def _matmul_body(x_ref, w_ref, out_ref):
    out_ref[...] = jnp.dot(x_ref[...], w_ref[...], preferred_element_type=jnp.float32)


def _matmul(x, w):
    grid = _NODES // _BM
    return pl.pallas_call(
        _matmul_body,
        grid=(grid,),
        in_specs=[
            pl.BlockSpec((_BM, _D), lambda i: (i, 0)),
            pl.BlockSpec((_D, _D), lambda i: (0, 0)),
        ],
        out_specs=pl.BlockSpec((_BM, _D), lambda i: (i, 0)),
        out_shape=jax.ShapeDtypeStruct((_NODES, _D), jnp.float32),
    )(x, w)


def _out_norm_body(p_ref, cinv_ref, yr_ref, wl_ref, bl_ref, out_ref):
    mean = (p_ref[...].astype(jnp.float32) * cinv_ref[...]).astype(jnp.bfloat16)
    out = (
        jnp.dot(mean, wl_ref[...], preferred_element_type=jnp.float32)
        + bl_ref[...]
        + yr_ref[...]
    )
    ssq = jnp.sum(out * out, axis=-1, keepdims=True)
    out_ref[...] = (out / jnp.maximum(jnp.sqrt(ssq), 1e-12)).astype(out_ref.dtype)


def _out_norm(part, cnt_inv, yr, wl, bl, out_dtype):
    grid = _NODES // _BM
    return pl.pallas_call(
        _out_norm_body,
        grid=(grid,),
        in_specs=[
            pl.BlockSpec((_BM, _D), lambda i: (i, 0)),
            pl.BlockSpec((_BM, 1), lambda i: (i, 0)),
            pl.BlockSpec((_BM, _D), lambda i: (i, 0)),
            pl.BlockSpec((_D, _D), lambda i: (0, 0)),
            pl.BlockSpec((1, _D), lambda i: (0, 0)),
        ],
        out_specs=pl.BlockSpec((_BM, _D), lambda i: (i, 0)),
        out_shape=jax.ShapeDtypeStruct((_NODES, _D), out_dtype),
    )(part, cnt_inv, yr, wl, bl.reshape(1, _D))


_SC_MESH = plsc.VectorSubcoreMesh(core_axis_name="c", subcore_axis_name="s")
_SC_PARAMS = pltpu.CompilerParams(use_tc_tiling_on_sc=False,
                                  needs_layout_passes=False)


def _prep_body(src_hbm, dst_hbm, cnt_hbm, glist_hbm, nch_hbm,
               src_v, dst_v, hist_v, lsrc_v, ldst_v, cnts_v, cnt_sh, sem):
    c = lax.axis_index("c")
    s = lax.axis_index("s")
    wid = s * _NC + c
    pltpu.sync_copy(src_hbm.at[pl.ds(wid * _EPT, _EPT)], src_v.at[pl.ds(0, _EPT)])
    pltpu.sync_copy(dst_hbm.at[pl.ds(wid * _EPT, _EPT)], dst_v.at[pl.ds(0, _EPT)])

    zero16 = jnp.zeros((16,), jnp.float32)
    lanes = lax.iota(jnp.int32, 16)

    def zero_hist(t, carry):
        hist_v[t // 8, pl.ds((t % 8) * 16, 16)] = zero16
        return carry
    lax.fori_loop(0, _HISTR * 8, zero_hist, 0)

    @pl.when(s < 10)
    def _zero_sh():
        pltpu.sync_copy(hist_v.at[pl.ds(s * 8, 8)], cnt_sh.at[pl.ds(s * 8, 8)])
    plsc.subcore_barrier()

    ones = jnp.ones((16,), jnp.float32)

    def count(i, carry):
        nlanes = jnp.where(i == _NCHUNK, 8, 16)
        m = lanes < nlanes
        d = dst_v[pl.ds(i * 16, 16)]
        plsc.addupdate_scatter(hist_v, [lax.shift_right_logical(d, 7), d & 127],
                               ones, mask=m)
        return carry
    lax.fori_loop(0, _NCHUNK + 1, count, 0)

    for k in range(_HISTR // 16):
        idxv = lax.iota(jnp.int32, 16) + k * 16
        pltpu.sync_copy(hist_v.at[pl.ds(k * 16, 16)], cnt_sh.at[idxv], add=True)
    plsc.subcore_barrier()

    @pl.when(s < 10)
    def _writeback():
        pltpu.sync_copy(cnt_sh.at[pl.ds(s * 8, 8)], cnt_hbm.at[c, pl.ds(s * 8, 8)])

    def filt(i, ns):
        nlanes = jnp.where(i == _NCHUNK, 8, 16)
        m = lanes < nlanes
        d = dst_v[pl.ds(i * 16, 16)]
        sv = src_v[pl.ds(i * 16, 16)]
        out = []
        for q in range(_NBKT):
            inb = m & (d >= q * _BKT) & (d < (q + 1) * _BKT)
            plsc.store_compressed(lsrc_v.at[pl.ds(q * _CAP + ns[q], 16)], sv, mask=inb)
            plsc.store_compressed(ldst_v.at[pl.ds(q * _CAP + ns[q], 16)], d - q * _BKT,
                                  mask=inb)
            out.append(ns[q] + jnp.sum(inb.astype(jnp.int32)))
        return tuple(out)
    ns = lax.fori_loop(0, _NCHUNK + 1, filt, (0,) * _NBKT)

    trash16 = jnp.full((16,), _TRASH, jnp.int32)
    big16 = jnp.full((16,), _BIGSRC, jnp.int32)
    for q in range(_NBKT):
        for t in range(2):
            lsrc_v[pl.ds(q * _CAP + ns[q] + t * 16, 16)] = big16
            ldst_v[pl.ds(q * _CAP + ns[q] + t * 16, 16)] = trash16
        n1ch = (ns[q] + 15) // 16
        cnts_v[pl.ds(q * 16, 16)] = jnp.full((16,), n1ch, jnp.int32)
        pltpu.sync_copy(lsrc_v.at[pl.ds(q * _CAP, _CAP)], glist_hbm.at[wid, q, 0])
        pltpu.sync_copy(ldst_v.at[pl.ds(q * _CAP, _CAP)], glist_hbm.at[wid, q, 1])
    pltpu.sync_copy(cnts_v, nch_hbm.at[wid])


def _sc_prep(src, dst):
    return pl.kernel(
        _prep_body,
        out_type=(
            jax.ShapeDtypeStruct((_NC, _HISTR, 128), jnp.float32),
            jax.ShapeDtypeStruct((_NW, _NBKT, 2, _CAP), jnp.int32),
            jax.ShapeDtypeStruct((_NW, _NBKT * 16), jnp.int32),
        ),
        mesh=_SC_MESH,
        compiler_params=_SC_PARAMS,
        scratch_types=[
            pltpu.VMEM((_EPT + 16,), jnp.int32),
            pltpu.VMEM((_EPT + 16,), jnp.int32),
            pltpu.VMEM((_HISTR, 128), jnp.float32),
            pltpu.VMEM((_NBKT * _CAP,), jnp.int32),
            pltpu.VMEM((_NBKT * _CAP,), jnp.int32),
            pltpu.VMEM((_NBKT * 16,), jnp.int32),
            pltpu.VMEM_SHARED((_HISTR, 128), jnp.float32),
            pltpu.SemaphoreType.DMA,
        ],
    )(src, dst)


def _segsum_body(xp_hbm, glist_hbm, nch_hbm, zrows_hbm, out_hbm,
                 l1_v, l2s_v, l2d_v, rows_a, rows_b, didx_a, didx_b,
                 cbuf_v, stage_sh, acc_sh, sem_a, sem_b):
    c = lax.axis_index("c")
    s = lax.axis_index("s")
    pltpu.sync_copy(nch_hbm.at[2 * s], cbuf_v.at[pl.ds(0, _NBKT * 16)])
    pltpu.sync_copy(nch_hbm.at[2 * s + 1], cbuf_v.at[pl.ds(_NBKT * 16, _NBKT * 16)])

    lanes = lax.iota(jnp.int32, 16)
    lane0 = (lanes == 0).astype(jnp.int32)
    trash16 = jnp.full((16,), _TRASH, jnp.int32)
    zero16i = jnp.zeros((16,), jnp.int32)

    def bucket(qq, carry_q):
        q = c * (_NBKT // _NC) + qq
        lo = q * _BKT

        pltpu.sync_copy(glist_hbm.at[2 * s, q], l1_v.at[0])
        pltpu.sync_copy(glist_hbm.at[2 * s + 1, q], l1_v.at[1])

        for t in range(8):
            pltpu.sync_copy(zrows_hbm, acc_sh.at[pl.ds(s * 64 + t * 8, 8)])
        plsc.subcore_barrier()

        def block(b, carry_b):
            @pl.when(s < 15)
            def _stage():
                pltpu.sync_copy(xp_hbm.at[pl.ds(b * _BLK + s * 128, 128)],
                                stage_sh.at[pl.ds(s * 128, 128)])

            @pl.when(s == 15)
            def _stage_tail():
                pltpu.sync_copy(xp_hbm.at[pl.ds(b * _BLK + 1920, 80)],
                                stage_sh.at[pl.ds(1920, 80)])
            plsc.subcore_barrier()

            n2 = 0
            for pi in range(2):
                n1ch = jnp.sum(cbuf_v[pl.ds(pi * _NBKT * 16 + q * 16, 16)] * lane0)

                def filt(i, n):
                    sv = l1_v[pi, 0, pl.ds(i * 16, 16)]
                    dv = l1_v[pi, 1, pl.ds(i * 16, 16)]
                    inb = (sv >= b * _BLK) & (sv < (b + 1) * _BLK)
                    plsc.store_compressed(l2s_v.at[pl.ds(n, 16)], sv - b * _BLK,
                                          mask=inb)
                    plsc.store_compressed(l2d_v.at[pl.ds(n, 16)], dv, mask=inb)
                    return n + jnp.sum(inb.astype(jnp.int32))
                n2 = lax.fori_loop(0, n1ch, filt, n2)

            for t in range(5):
                l2s_v[pl.ds(n2 + t * 16, 16)] = zero16i
                l2d_v[pl.ds(n2 + t * 16, 16)] = trash16
            nch2 = (n2 + _ROWS - 1) // _ROWS

            @pl.when(nch2 > 0)
            def _prime0():
                pltpu.async_copy(stage_sh.at[l2s_v.at[pl.ds(0, _ROWS)]],
                                 rows_a, sem_a)

            @pl.when(nch2 > 1)
            def _prime1():
                pltpu.async_copy(stage_sh.at[l2s_v.at[pl.ds(_ROWS, _ROWS)]],
                                 rows_b, sem_b)

            def chunk(j, carry):
                nxt = j + 2

                @pl.when(j % 2 == 0)
                def _even():
                    pltpu.make_async_copy(
                        stage_sh.at[l2s_v.at[pl.ds(j * _ROWS, _ROWS)]],
                        rows_a, sem_a).wait()
                    for k in range(_ROWS // 16):
                        didx_a[pl.ds(k * 16, 16)] = l2d_v[pl.ds(j * _ROWS + k * 16, 16)]
                    pltpu.sync_copy(rows_a, acc_sh.at[didx_a], add=True)

                    @pl.when(nxt < nch2)
                    def _issue():
                        pltpu.async_copy(
                            stage_sh.at[l2s_v.at[pl.ds(nxt * _ROWS, _ROWS)]],
                            rows_a, sem_a)

                @pl.when(j % 2 == 1)
                def _odd():
                    pltpu.make_async_copy(
                        stage_sh.at[l2s_v.at[pl.ds(j * _ROWS, _ROWS)]],
                        rows_b, sem_b).wait()
                    for k in range(_ROWS // 16):
                        didx_b[pl.ds(k * 16, 16)] = l2d_v[pl.ds(j * _ROWS + k * 16, 16)]
                    pltpu.sync_copy(rows_b, acc_sh.at[didx_b], add=True)

                    @pl.when(nxt < nch2)
                    def _issue():
                        pltpu.async_copy(
                            stage_sh.at[l2s_v.at[pl.ds(nxt * _ROWS, _ROWS)]],
                            rows_b, sem_b)
                return carry
            lax.fori_loop(0, nch2, chunk, 0)
            plsc.subcore_barrier()
            return carry_b
        lax.fori_loop(0, _NBLK, block, 0)

        @pl.when(s < 15)
        def _wb():
            pltpu.sync_copy(acc_sh.at[pl.ds(s * 64, 64)],
                            out_hbm.at[pl.ds(lo + s * 64, 64)])

        @pl.when(s == 15)
        def _wb_tail():
            pltpu.sync_copy(acc_sh.at[pl.ds(960, 40)],
                            out_hbm.at[pl.ds(lo + 960, 40)])
        plsc.subcore_barrier()
        return carry_q
    lax.fori_loop(0, _NBKT // _NC, bucket, 0)


def _sc_segsum(xp, glist, nch, zrows):
    return pl.kernel(
        _segsum_body,
        out_type=jax.ShapeDtypeStruct((_NODES, _D), jnp.bfloat16),
        mesh=_SC_MESH,
        compiler_params=_SC_PARAMS,
        scratch_types=[
            pltpu.VMEM((2, 2, _CAP), jnp.int32),
            pltpu.VMEM((_L2CAP,), jnp.int32),
            pltpu.VMEM((_L2CAP,), jnp.int32),
            pltpu.VMEM((_ROWS, _D), jnp.bfloat16),
            pltpu.VMEM((_ROWS, _D), jnp.bfloat16),
            pltpu.VMEM((_ROWS,), jnp.int32),
            pltpu.VMEM((_ROWS,), jnp.int32),
            pltpu.VMEM((2 * _NBKT * 16,), jnp.int32),
            pltpu.VMEM_SHARED((_BLK, _D), jnp.bfloat16),
            pltpu.VMEM_SHARED((_ACCR, _D), jnp.bfloat16),
            pltpu.SemaphoreType.DMA,
            pltpu.SemaphoreType.DMA,
        ],
    )(xp, glist, nch, zrows)


def kernel(svg_path, svg_path_mask, edge_index, type_embed, coor_embed, W_in, b_in,
           W1p, b1p, W1l, b1l, W1r, W2p, b2p, W2l, b2l, W2r):
    svg = jnp.where(svg_path_mask, svg_path, 0)
    cmd_idx = svg[:, :, 0]
    coor_idx = svg[:, :, 1:] + 3
    idx_all = jnp.concatenate(
        [cmd_idx.reshape(_B, _N), coor_idx.reshape(_B, _N * (_C - 1))], axis=1
    ).reshape(_NODES)
    table = jnp.concatenate(
        [type_embed, coor_embed,
         jnp.zeros((_TPAD - 3 - 200, _D), jnp.float32)], axis=0
    ).astype(jnp.bfloat16)
    W_in_b = W_in.astype(jnp.bfloat16)
    W1p_b, W1l_b, W1r_b = (W1p.astype(jnp.bfloat16), W1l.astype(jnp.bfloat16),
                           W1r.astype(jnp.bfloat16))
    W2p_b, W2l_b, W2r_b = (W2p.astype(jnp.bfloat16), W2l.astype(jnp.bfloat16),
                           W2r.astype(jnp.bfloat16))
    src = edge_index[0]
    dst = edge_index[1]
    zrows = jnp.zeros((8, _D), jnp.bfloat16)

    cntp, glistw, nchw = _sc_prep(src, dst)
    cnt = (cntp[0] + cntp[1]).reshape(_HISTR * 128)[:_NODES]
    cnt_inv = (1.0 / jnp.maximum(cnt, 1.0)).reshape(_NODES, 1)

    x = _embed_matmul(idx_all, table, W_in_b, b_in.reshape(1, _D))

    xp1 = _relu_matmul(x, W1p_b, b1p)
    yr1 = _matmul(xp1, W1r_b)
    part1 = _sc_segsum(xp1, glistw, nchw, zrows)
    x1 = _out_norm(part1, cnt_inv, yr1, W1l_b, b1l, jnp.bfloat16)

    xp2 = _relu_matmul(x1, W2p_b, b2p)
    yr2 = _matmul(xp2, W2r_b)
    part2 = _sc_segsum(xp2, glistw, nchw, zrows)
    x2 = _out_norm(part2, cnt_inv, yr2, W2l_b, b2l, jnp.float32)
    return x2


# speedup vs baseline: 1.6827x; 1.0196x over previous
import jax
import jax.numpy as jnp
from jax import lax
from jax.experimental import pallas as pl
from jax.experimental.pallas import tpu as pltpu
from jax.experimental.pallas import tpu_sc as plsc

_B, _N, _C = 2, 1250, 4
_D = 512
_NODES = _B * _N * _C
_E = 160000
_BM = 2000
_TPAD = 256

_NC = 2
_NS = 16
_NW = _NC * _NS
_EPT = _E // _NW
_NCHUNK = _EPT // 16
_NBKT = 10
_BKT = _NODES // _NBKT
_NBLK = 5
_BLK = 2000
_ACCR = 1024
_TRASH = 1016
_ROWS = 64
_CAP = 5120
_L2CAP = 10176
_BIGSRC = 1 << 20
_HISTR = 80


def _k1_body(idx_ref, table_ref, win_ref, bin_ref, w1p_ref, b1p_ref, w1r_ref,
             xp_ref, yr_ref):
    idx = idx_ref[0, 0, :]
    onehot = (idx[:, None] == lax.broadcasted_iota(jnp.int32, (_BM, _TPAD), 1)).astype(jnp.bfloat16)
    embed = jnp.dot(onehot, table_ref[...], preferred_element_type=jnp.float32)
    h = jnp.dot(embed.astype(jnp.bfloat16), win_ref[...],
                preferred_element_type=jnp.float32) + bin_ref[...]
    xp = jax.nn.relu(
        jnp.dot(h.astype(jnp.bfloat16), w1p_ref[...],
                preferred_element_type=jnp.float32) + b1p_ref[...]
    ).astype(jnp.bfloat16)
    xp_ref[...] = xp
    yr_ref[...] = jnp.dot(xp, w1r_ref[...], preferred_element_type=jnp.float32)


def _k1(idx_all, table, win, bin_, w1p, b1p, w1r):
    grid = _NODES // _BM
    idx3 = idx_all.reshape(grid, 1, _BM)
    full = lambda i: (0, 0)
    return pl.pallas_call(
        _k1_body,
        grid=(grid,),
        in_specs=[
            pl.BlockSpec((1, 1, _BM), lambda i: (i, 0, 0)),
            pl.BlockSpec((_TPAD, _D), full),
            pl.BlockSpec((_D, _D), full),
            pl.BlockSpec((1, _D), full),
            pl.BlockSpec((_D, _D), full),
            pl.BlockSpec((1, _D), full),
            pl.BlockSpec((_D, _D), full),
        ],
        out_specs=[
            pl.BlockSpec((_BM, _D), lambda i: (i, 0)),
            pl.BlockSpec((_BM, _D), lambda i: (i, 0)),
        ],
        out_shape=[
            jax.ShapeDtypeStruct((_NODES, _D), jnp.bfloat16),
            jax.ShapeDtypeStruct((_NODES, _D), jnp.float32),
        ],
    )(idx3, table, win, bin_.reshape(1, _D), w1p, b1p.reshape(1, _D), w1r)


def _norm_from(p_ref, cinv_ref, yr_ref, wl_ref, bl_ref):
    mean = (p_ref[...].astype(jnp.float32) * cinv_ref[...]).astype(jnp.bfloat16)
    out = (
        jnp.dot(mean, wl_ref[...], preferred_element_type=jnp.float32)
        + bl_ref[...]
        + yr_ref[...]
    )
    ssq = jnp.sum(out * out, axis=-1, keepdims=True)
    return out / jnp.maximum(jnp.sqrt(ssq), 1e-12)


def _k2_body(p_ref, cinv_ref, yr_ref, wl_ref, bl_ref, wp_ref, bp_ref, wr_ref,
             xp_ref, yr2_ref):
    x1 = _norm_from(p_ref, cinv_ref, yr_ref, wl_ref, bl_ref).astype(jnp.bfloat16)
    xp = jax.nn.relu(
        jnp.dot(x1, wp_ref[...], preferred_element_type=jnp.float32) + bp_ref[...]
    ).astype(jnp.bfloat16)
    xp_ref[...] = xp
    yr2_ref[...] = jnp.dot(xp, wr_ref[...], preferred_element_type=jnp.float32)


def _k2(part, cnt_inv, yr, wl, bl, wp, bp, wr):
    grid = _NODES // _BM
    full = lambda i: (0, 0)
    return pl.pallas_call(
        _k2_body,
        grid=(grid,),
        in_specs=[
            pl.BlockSpec((_BM, _D), lambda i: (i, 0)),
            pl.BlockSpec((_BM, 1), lambda i: (i, 0)),
            pl.BlockSpec((_BM, _D), lambda i: (i, 0)),
            pl.BlockSpec((_D, _D), full),
            pl.BlockSpec((1, _D), full),
            pl.BlockSpec((_D, _D), full),
            pl.BlockSpec((1, _D), full),
            pl.BlockSpec((_D, _D), full),
        ],
        out_specs=[
            pl.BlockSpec((_BM, _D), lambda i: (i, 0)),
            pl.BlockSpec((_BM, _D), lambda i: (i, 0)),
        ],
        out_shape=[
            jax.ShapeDtypeStruct((_NODES, _D), jnp.bfloat16),
            jax.ShapeDtypeStruct((_NODES, _D), jnp.float32),
        ],
    )(part, cnt_inv, yr, wl, bl.reshape(1, _D), wp, bp.reshape(1, _D), wr)


def _k3_body(p_ref, cinv_ref, yr_ref, wl_ref, bl_ref, out_ref):
    out_ref[...] = _norm_from(p_ref, cinv_ref, yr_ref, wl_ref, bl_ref)


def _k3(part, cnt_inv, yr, wl, bl):
    grid = _NODES // _BM
    full = lambda i: (0, 0)
    return pl.pallas_call(
        _k3_body,
        grid=(grid,),
        in_specs=[
            pl.BlockSpec((_BM, _D), lambda i: (i, 0)),
            pl.BlockSpec((_BM, 1), lambda i: (i, 0)),
            pl.BlockSpec((_BM, _D), lambda i: (i, 0)),
            pl.BlockSpec((_D, _D), full),
            pl.BlockSpec((1, _D), full),
        ],
        out_specs=pl.BlockSpec((_BM, _D), lambda i: (i, 0)),
        out_shape=jax.ShapeDtypeStruct((_NODES, _D), jnp.float32),
    )(part, cnt_inv, yr, wl, bl.reshape(1, _D))


_SC_MESH = plsc.VectorSubcoreMesh(core_axis_name="c", subcore_axis_name="s")
_SC_PARAMS = pltpu.CompilerParams(use_tc_tiling_on_sc=False,
                                  needs_layout_passes=False)


def _prep_body(src_hbm, dst_hbm, cnt_hbm, glist_hbm, nch_hbm,
               src_v, dst_v, hist_v, lsrc_v, ldst_v, cnts_v, cnt_sh, sem):
    c = lax.axis_index("c")
    s = lax.axis_index("s")
    wid = s * _NC + c
    pltpu.sync_copy(src_hbm.at[pl.ds(wid * _EPT, _EPT)], src_v.at[pl.ds(0, _EPT)])
    pltpu.sync_copy(dst_hbm.at[pl.ds(wid * _EPT, _EPT)], dst_v.at[pl.ds(0, _EPT)])

    zero16 = jnp.zeros((16,), jnp.float32)
    lanes = lax.iota(jnp.int32, 16)

    def zero_hist(t, carry):
        hist_v[t // 8, pl.ds((t % 8) * 16, 16)] = zero16
        return carry
    lax.fori_loop(0, _HISTR * 8, zero_hist, 0)

    @pl.when(s < 10)
    def _zero_sh():
        pltpu.sync_copy(hist_v.at[pl.ds(s * 8, 8)], cnt_sh.at[pl.ds(s * 8, 8)])
    plsc.subcore_barrier()

    ones = jnp.ones((16,), jnp.float32)

    def count(i, carry):
        nlanes = jnp.where(i == _NCHUNK, 8, 16)
        m = lanes < nlanes
        d = dst_v[pl.ds(i * 16, 16)]
        plsc.addupdate_scatter(hist_v, [lax.shift_right_logical(d, 7), d & 127],
                               ones, mask=m)
        return carry
    lax.fori_loop(0, _NCHUNK + 1, count, 0)

    for k in range(_HISTR // 16):
        idxv = lax.iota(jnp.int32, 16) + k * 16
        pltpu.sync_copy(hist_v.at[pl.ds(k * 16, 16)], cnt_sh.at[idxv], add=True)
    plsc.subcore_barrier()

    @pl.when(s < 10)
    def _writeback():
        pltpu.sync_copy(cnt_sh.at[pl.ds(s * 8, 8)], cnt_hbm.at[c, pl.ds(s * 8, 8)])

    def filt(i, ns):
        nlanes = jnp.where(i == _NCHUNK, 8, 16)
        m = lanes < nlanes
        d = dst_v[pl.ds(i * 16, 16)]
        sv = src_v[pl.ds(i * 16, 16)]
        out = []
        for q in range(_NBKT):
            inb = m & (d >= q * _BKT) & (d < (q + 1) * _BKT)
            plsc.store_compressed(lsrc_v.at[pl.ds(q * _CAP + ns[q], 16)], sv, mask=inb)
            plsc.store_compressed(ldst_v.at[pl.ds(q * _CAP + ns[q], 16)], d - q * _BKT,
                                  mask=inb)
            out.append(ns[q] + jnp.sum(inb.astype(jnp.int32)))
        return tuple(out)
    ns = lax.fori_loop(0, _NCHUNK + 1, filt, (0,) * _NBKT)

    trash16 = jnp.full((16,), _TRASH, jnp.int32)
    big16 = jnp.full((16,), _BIGSRC, jnp.int32)
    for q in range(_NBKT):
        for t in range(2):
            lsrc_v[pl.ds(q * _CAP + ns[q] + t * 16, 16)] = big16
            ldst_v[pl.ds(q * _CAP + ns[q] + t * 16, 16)] = trash16
        n1ch = (ns[q] + 15) // 16
        cnts_v[pl.ds(q * 16, 16)] = jnp.full((16,), n1ch, jnp.int32)
        pltpu.sync_copy(lsrc_v.at[pl.ds(q * _CAP, _CAP)], glist_hbm.at[wid, q, 0])
        pltpu.sync_copy(ldst_v.at[pl.ds(q * _CAP, _CAP)], glist_hbm.at[wid, q, 1])
    pltpu.sync_copy(cnts_v, nch_hbm.at[wid])


def _sc_prep(src, dst):
    return pl.kernel(
        _prep_body,
        out_type=(
            jax.ShapeDtypeStruct((_NC, _HISTR, 128), jnp.float32),
            jax.ShapeDtypeStruct((_NW, _NBKT, 2, _CAP), jnp.int32),
            jax.ShapeDtypeStruct((_NW, _NBKT * 16), jnp.int32),
        ),
        mesh=_SC_MESH,
        compiler_params=_SC_PARAMS,
        scratch_types=[
            pltpu.VMEM((_EPT + 16,), jnp.int32),
            pltpu.VMEM((_EPT + 16,), jnp.int32),
            pltpu.VMEM((_HISTR, 128), jnp.float32),
            pltpu.VMEM((_NBKT * _CAP,), jnp.int32),
            pltpu.VMEM((_NBKT * _CAP,), jnp.int32),
            pltpu.VMEM((_NBKT * 16,), jnp.int32),
            pltpu.VMEM_SHARED((_HISTR, 128), jnp.float32),
            pltpu.SemaphoreType.DMA,
        ],
    )(src, dst)


def _segsum_body(xp_hbm, glist_hbm, nch_hbm, zrows_hbm, out_hbm,
                 l1_v, l2s_v, l2d_v, rows_a, rows_b, didx_a, didx_b,
                 cbuf_v, stage_sh, acc_sh, sem_a, sem_b):
    c = lax.axis_index("c")
    s = lax.axis_index("s")
    pltpu.sync_copy(nch_hbm.at[2 * s], cbuf_v.at[pl.ds(0, _NBKT * 16)])
    pltpu.sync_copy(nch_hbm.at[2 * s + 1], cbuf_v.at[pl.ds(_NBKT * 16, _NBKT * 16)])

    lanes = lax.iota(jnp.int32, 16)
    lane0 = (lanes == 0).astype(jnp.int32)
    trash16 = jnp.full((16,), _TRASH, jnp.int32)
    zero16i = jnp.zeros((16,), jnp.int32)

    def bucket(qq, carry_q):
        q = c * (_NBKT // _NC) + qq
        lo = q * _BKT

        pltpu.sync_copy(glist_hbm.at[2 * s, q], l1_v.at[0])
        pltpu.sync_copy(glist_hbm.at[2 * s + 1, q], l1_v.at[1])

        for t in range(8):
            pltpu.sync_copy(zrows_hbm, acc_sh.at[pl.ds(s * 64 + t * 8, 8)])
        plsc.subcore_barrier()

        def block(b, carry_b):
            @pl.when(s < 15)
            def _stage():
                pltpu.sync_copy(xp_hbm.at[pl.ds(b * _BLK + s * 128, 128)],
                                stage_sh.at[pl.ds(s * 128, 128)])

            @pl.when(s == 15)
            def _stage_tail():
                pltpu.sync_copy(xp_hbm.at[pl.ds(b * _BLK + 1920, 80)],
                                stage_sh.at[pl.ds(1920, 80)])
            plsc.subcore_barrier()

            n2 = 0
            for pi in range(2):
                n1ch = jnp.sum(cbuf_v[pl.ds(pi * _NBKT * 16 + q * 16, 16)] * lane0)

                def filt(i, n):
                    sv = l1_v[pi, 0, pl.ds(i * 16, 16)]
                    dv = l1_v[pi, 1, pl.ds(i * 16, 16)]
                    inb = (sv >= b * _BLK) & (sv < (b + 1) * _BLK)
                    plsc.store_compressed(l2s_v.at[pl.ds(n, 16)], sv - b * _BLK,
                                          mask=inb)
                    plsc.store_compressed(l2d_v.at[pl.ds(n, 16)], dv, mask=inb)
                    return n + jnp.sum(inb.astype(jnp.int32))
                n2 = lax.fori_loop(0, n1ch, filt, n2)

            for t in range(5):
                l2s_v[pl.ds(n2 + t * 16, 16)] = zero16i
                l2d_v[pl.ds(n2 + t * 16, 16)] = trash16
            nch2 = (n2 + _ROWS - 1) // _ROWS

            @pl.when(nch2 > 0)
            def _prime0():
                pltpu.async_copy(stage_sh.at[l2s_v.at[pl.ds(0, _ROWS)]],
                                 rows_a, sem_a)

            @pl.when(nch2 > 1)
            def _prime1():
                pltpu.async_copy(stage_sh.at[l2s_v.at[pl.ds(_ROWS, _ROWS)]],
                                 rows_b, sem_b)

            def chunk(j, carry):
                nxt = j + 2

                @pl.when(j % 2 == 0)
                def _even():
                    pltpu.make_async_copy(
                        stage_sh.at[l2s_v.at[pl.ds(j * _ROWS, _ROWS)]],
                        rows_a, sem_a).wait()
                    for k in range(_ROWS // 16):
                        didx_a[pl.ds(k * 16, 16)] = l2d_v[pl.ds(j * _ROWS + k * 16, 16)]
                    pltpu.sync_copy(rows_a, acc_sh.at[didx_a], add=True)

                    @pl.when(nxt < nch2)
                    def _issue():
                        pltpu.async_copy(
                            stage_sh.at[l2s_v.at[pl.ds(nxt * _ROWS, _ROWS)]],
                            rows_a, sem_a)

                @pl.when(j % 2 == 1)
                def _odd():
                    pltpu.make_async_copy(
                        stage_sh.at[l2s_v.at[pl.ds(j * _ROWS, _ROWS)]],
                        rows_b, sem_b).wait()
                    for k in range(_ROWS // 16):
                        didx_b[pl.ds(k * 16, 16)] = l2d_v[pl.ds(j * _ROWS + k * 16, 16)]
                    pltpu.sync_copy(rows_b, acc_sh.at[didx_b], add=True)

                    @pl.when(nxt < nch2)
                    def _issue():
                        pltpu.async_copy(
                            stage_sh.at[l2s_v.at[pl.ds(nxt * _ROWS, _ROWS)]],
                            rows_b, sem_b)
                return carry
            lax.fori_loop(0, nch2, chunk, 0)
            plsc.subcore_barrier()
            return carry_b
        lax.fori_loop(0, _NBLK, block, 0)

        @pl.when(s < 15)
        def _wb():
            pltpu.sync_copy(acc_sh.at[pl.ds(s * 64, 64)],
                            out_hbm.at[pl.ds(lo + s * 64, 64)])

        @pl.when(s == 15)
        def _wb_tail():
            pltpu.sync_copy(acc_sh.at[pl.ds(960, 40)],
                            out_hbm.at[pl.ds(lo + 960, 40)])
        plsc.subcore_barrier()
        return carry_q
    lax.fori_loop(0, _NBKT // _NC, bucket, 0)


def _sc_segsum(xp, glist, nch, zrows):
    return pl.kernel(
        _segsum_body,
        out_type=jax.ShapeDtypeStruct((_NODES, _D), jnp.bfloat16),
        mesh=_SC_MESH,
        compiler_params=_SC_PARAMS,
        scratch_types=[
            pltpu.VMEM((2, 2, _CAP), jnp.int32),
            pltpu.VMEM((_L2CAP,), jnp.int32),
            pltpu.VMEM((_L2CAP,), jnp.int32),
            pltpu.VMEM((_ROWS, _D), jnp.bfloat16),
            pltpu.VMEM((_ROWS, _D), jnp.bfloat16),
            pltpu.VMEM((_ROWS,), jnp.int32),
            pltpu.VMEM((_ROWS,), jnp.int32),
            pltpu.VMEM((2 * _NBKT * 16,), jnp.int32),
            pltpu.VMEM_SHARED((_BLK, _D), jnp.bfloat16),
            pltpu.VMEM_SHARED((_ACCR, _D), jnp.bfloat16),
            pltpu.SemaphoreType.DMA,
            pltpu.SemaphoreType.DMA,
        ],
    )(xp, glist, nch, zrows)


def kernel(svg_path, svg_path_mask, edge_index, type_embed, coor_embed, W_in, b_in,
           W1p, b1p, W1l, b1l, W1r, W2p, b2p, W2l, b2l, W2r):
    svg = jnp.where(svg_path_mask, svg_path, 0)
    cmd_idx = svg[:, :, 0]
    coor_idx = svg[:, :, 1:] + 3
    idx_all = jnp.concatenate(
        [cmd_idx.reshape(_B, _N), coor_idx.reshape(_B, _N * (_C - 1))], axis=1
    ).reshape(_NODES)
    table = jnp.concatenate(
        [type_embed, coor_embed,
         jnp.zeros((_TPAD - 3 - 200, _D), jnp.float32)], axis=0
    ).astype(jnp.bfloat16)
    W_in_b = W_in.astype(jnp.bfloat16)
    W1p_b, W1l_b, W1r_b = (W1p.astype(jnp.bfloat16), W1l.astype(jnp.bfloat16),
                           W1r.astype(jnp.bfloat16))
    W2p_b, W2l_b, W2r_b = (W2p.astype(jnp.bfloat16), W2l.astype(jnp.bfloat16),
                           W2r.astype(jnp.bfloat16))
    src = edge_index[0]
    dst = edge_index[1]
    zrows = jnp.zeros((8, _D), jnp.bfloat16)

    cntp, glistw, nchw = _sc_prep(src, dst)
    cnt = (cntp[0] + cntp[1]).reshape(_HISTR * 128)[:_NODES]
    cnt_inv = (1.0 / jnp.maximum(cnt, 1.0)).reshape(_NODES, 1)

    xp1, yr1 = _k1(idx_all, table, W_in_b, b_in, W1p_b, b1p, W1r_b)
    part1 = _sc_segsum(xp1, glistw, nchw, zrows)
    xp2, yr2 = _k2(part1, cnt_inv, yr1, W1l_b, b1l, W2p_b, b2p, W2r_b)
    part2 = _sc_segsum(xp2, glistw, nchw, zrows)
    x2 = _k3(part2, cnt_inv, yr2, W2l_b, b2l)
    return x2
